# Initial kernel scaffold; baseline (speedup 1.0000x reference)
import jax
import jax.numpy as jnp
from jax.experimental import pallas as pl


def kernel(slot_features, W, b_lin, embed):
    raise NotImplementedError("write your pallas kernel here")


# speedup vs baseline: 24.7069x; 24.7069x over previous
import functools

import jax
import jax.numpy as jnp
from jax import lax
from jax.experimental import pallas as pl
from jax.experimental.pallas import tpu as pltpu

N_CODES = 1024
CODE_DIM = 64
K_TOP = 8
INV_TEMP = 10.0


def _vq_kernel(x_ref, wt_ref, b_ref, e_ref,
               q_ref, idx_ref, loss_ref, ent_ref,
               usage_scr, loss_scr, *, nsteps, n_rows):
    i = pl.program_id(0)

    @pl.when(i == 0)
    def _init():
        usage_scr[...] = jnp.zeros_like(usage_scr)
        loss_scr[0, 0] = 0.0

    r = x_ref.shape[0]
    x = x_ref[...]
    z = lax.dot_general(x, wt_ref[...], (((1,), (0,)), ((), ())),
                        preferred_element_type=jnp.float32,
                        precision=lax.Precision.DEFAULT) + b_ref[...]
    e = e_ref[...]
    e2 = jnp.sum(e * e, axis=1)
    z2 = jnp.sum(z * z, axis=1, keepdims=True)
    cross = lax.dot_general(z, e, (((1,), (1,)), ((), ())),
                            preferred_element_type=jnp.float32,
                            precision=lax.Precision.DEFAULT)
    sq = z2 + e2[None, :] - 2.0 * cross

    BIG = jnp.float32(3e38)
    work = sq
    vals = []
    amin = None
    for j in range(K_TOP):
        m = jnp.min(work, axis=1, keepdims=True)
        vals.append(m)
        if j == 0:
            iota = lax.broadcasted_iota(jnp.int32, (r, N_CODES), 1)
            idxsel = jnp.where(work == m, iota, N_CODES)
            amin = jnp.min(idxsel, axis=1).astype(jnp.int32)
            idx_ref[...] = amin
        if j < K_TOP - 1:
            work = jnp.where(work == m, BIG, work)

    m_stack = jnp.concatenate(vals, axis=1)
    d_vals = jnp.sqrt(jnp.clip(m_stack, 1e-12, None))
    d0 = d_vals[:, 0:1]
    wexp = jnp.exp((d0 - d_vals) * INV_TEMP)
    denom = jnp.sum(wexp, axis=1, keepdims=True)
    thresh = m_stack[:, K_TOP - 1:K_TOP]

    dist = jnp.sqrt(jnp.clip(sq, 1e-12, None))
    sw = jnp.where(sq <= thresh,
                   jnp.exp((d0 - dist) * INV_TEMP), 0.0) / denom

    q = lax.dot_general(sw, e, (((1,), (0,)), ((), ())),
                        preferred_element_type=jnp.float32,
                        precision=lax.Precision.DEFAULT)
    q_ref[...] = q

    usage_scr[...] += jnp.sum(sw, axis=0)
    loss_scr[0, 0] += jnp.sum((z - q) ** 2)

    @pl.when(i == nsteps - 1)
    def _fini():
        loss = loss_scr[0, 0] / jnp.float32(n_rows * CODE_DIM)
        loss_ref[...] = jnp.full((1, 1), loss, jnp.float32)
        avg = usage_scr[...] * jnp.float32(1.0 / n_rows)
        ent = -jnp.sum(avg * jnp.log(avg + 1e-8))
        ent_ref[...] = jnp.full((1, 1), ent, jnp.float32)


def kernel(slot_features, W, b_lin, embed):
    b, k, d_model = slot_features.shape
    n = b * k
    x = slot_features.reshape(n, d_model)
    wt = W.T
    bb = b_lin.reshape(1, CODE_DIM)

    r = 512
    while n % r:
        r //= 2
    nsteps = n // r

    q_flat, idx, loss, ent = pl.pallas_call(
        functools.partial(_vq_kernel, nsteps=nsteps, n_rows=n),
        grid=(nsteps,),
        in_specs=[
            pl.BlockSpec((r, d_model), lambda i: (i, 0)),
            pl.BlockSpec((d_model, CODE_DIM), lambda i: (0, 0)),
            pl.BlockSpec((1, CODE_DIM), lambda i: (0, 0)),
            pl.BlockSpec((N_CODES, CODE_DIM), lambda i: (0, 0)),
        ],
        out_specs=[
            pl.BlockSpec((r, CODE_DIM), lambda i: (i, 0)),
            pl.BlockSpec((r,), lambda i: (i,)),
            pl.BlockSpec((1, 1), lambda i: (0, 0)),
            pl.BlockSpec((1, 1), lambda i: (0, 0)),
        ],
        out_shape=[
            jax.ShapeDtypeStruct((n, CODE_DIM), jnp.float32),
            jax.ShapeDtypeStruct((n,), jnp.int32),
            jax.ShapeDtypeStruct((1, 1), jnp.float32),
            jax.ShapeDtypeStruct((1, 1), jnp.float32),
        ],
        scratch_shapes=[
            pltpu.VMEM((N_CODES,), jnp.float32),
            pltpu.SMEM((1, 1), jnp.float32),
        ],
        compiler_params=pltpu.CompilerParams(
            dimension_semantics=("arbitrary",)),
    )(x, wt, bb, embed)

    q_st = q_flat.reshape(b, k, CODE_DIM)
    indices = idx.reshape(b, k)
    return (q_st, indices, loss.reshape(()), ent.reshape(()))


# speedup vs baseline: 26.4959x; 1.0724x over previous
import functools

import jax
import jax.numpy as jnp
from jax import lax
from jax.experimental import pallas as pl
from jax.experimental.pallas import tpu as pltpu

N_CODES = 1024
CODE_DIM = 64
K_TOP = 8
INV_TEMP = 10.0


def _vq_kernel(x_ref, wt_ref, b_ref, e_ref, iota_ref,
               q_ref, idx_ref, loss_ref, ent_ref,
               usage_scr, loss_scr, *, nsteps, n_rows):
    i = pl.program_id(0)

    @pl.when(i == 0)
    def _init():
        usage_scr[...] = jnp.zeros_like(usage_scr)
        loss_scr[0, 0] = 0.0

    r = x_ref.shape[0]
    x = x_ref[...]
    z = lax.dot_general(x, wt_ref[...], (((1,), (0,)), ((), ())),
                        preferred_element_type=jnp.float32,
                        precision=lax.Precision.DEFAULT) + b_ref[...]
    e = e_ref[...]
    e2 = jnp.sum(e * e, axis=1)
    z2 = jnp.sum(z * z, axis=1, keepdims=True)
    cross = lax.dot_general(z, e, (((1,), (1,)), ((), ())),
                            preferred_element_type=jnp.float32,
                            precision=lax.Precision.DEFAULT)
    sq = z2 + e2[None, :] - 2.0 * cross

    BIG = jnp.float32(3e38)
    work = sq
    vals = []
    for j in range(K_TOP):
        m = jnp.min(work, axis=1, keepdims=True)
        vals.append(m)
        if j == 0:
            mask0 = jnp.where(work == m, 1.0, 0.0)
            idxf = lax.dot_general(mask0, iota_ref[...],
                                   (((1,), (0,)), ((), ())),
                                   preferred_element_type=jnp.float32)
            idx_ref[...] = idxf.astype(jnp.int32)
        if j < K_TOP - 1:
            work = jnp.where(work == m, BIG, work)

    m_stack = jnp.concatenate(vals, axis=1)
    d_vals = jnp.sqrt(jnp.clip(m_stack, 1e-12, None))
    d0 = d_vals[:, 0:1]
    wexp = jnp.exp((d0 - d_vals) * INV_TEMP)
    denom = jnp.sum(wexp, axis=1, keepdims=True)
    thresh = m_stack[:, K_TOP - 1:K_TOP]

    dist = jnp.sqrt(jnp.clip(sq, 1e-12, None))
    sw = jnp.where(sq <= thresh,
                   jnp.exp((d0 - dist) * INV_TEMP), 0.0) / denom

    q = lax.dot_general(sw, e, (((1,), (0,)), ((), ())),
                        preferred_element_type=jnp.float32,
                        precision=lax.Precision.DEFAULT)
    q_ref[...] = q

    usage_scr[...] += jnp.sum(sw, axis=0)
    loss_scr[0, 0] += jnp.sum((z - q) ** 2)

    @pl.when(i == nsteps - 1)
    def _fini():
        loss = loss_scr[0, 0] / jnp.float32(n_rows * CODE_DIM)
        loss_ref[...] = jnp.full((1, 1), loss, jnp.float32)
        avg = usage_scr[...] * jnp.float32(1.0 / n_rows)
        ent = -jnp.sum(avg * jnp.log(avg + 1e-8))
        ent_ref[...] = jnp.full((1, 1), ent, jnp.float32)


def kernel(slot_features, W, b_lin, embed):
    b, k, d_model = slot_features.shape
    n = b * k
    x = slot_features.reshape(n, d_model)
    wt = W.T
    bb = b_lin.reshape(1, CODE_DIM)

    r = 512
    while n % r:
        r //= 2
    nsteps = n // r

    q_flat, idx, loss, ent = pl.pallas_call(
        functools.partial(_vq_kernel, nsteps=nsteps, n_rows=n),
        grid=(nsteps,),
        in_specs=[
            pl.BlockSpec((r, d_model), lambda i: (i, 0)),
            pl.BlockSpec((d_model, CODE_DIM), lambda i: (0, 0)),
            pl.BlockSpec((1, CODE_DIM), lambda i: (0, 0)),
            pl.BlockSpec((N_CODES, CODE_DIM), lambda i: (0, 0)),
            pl.BlockSpec((N_CODES, 1), lambda i: (0, 0)),
        ],
        out_specs=[
            pl.BlockSpec((r, CODE_DIM), lambda i: (i, 0)),
            pl.BlockSpec((r, 1), lambda i: (i, 0)),
            pl.BlockSpec((1, 1), lambda i: (0, 0)),
            pl.BlockSpec((1, 1), lambda i: (0, 0)),
        ],
        out_shape=[
            jax.ShapeDtypeStruct((n, CODE_DIM), jnp.float32),
            jax.ShapeDtypeStruct((n, 1), jnp.int32),
            jax.ShapeDtypeStruct((1, 1), jnp.float32),
            jax.ShapeDtypeStruct((1, 1), jnp.float32),
        ],
        scratch_shapes=[
            pltpu.VMEM((N_CODES,), jnp.float32),
            pltpu.SMEM((1, 1), jnp.float32),
        ],
        compiler_params=pltpu.CompilerParams(
            dimension_semantics=("arbitrary",)),
    )(x, wt, bb, embed,
      jnp.arange(N_CODES, dtype=jnp.float32).reshape(N_CODES, 1))

    q_st = q_flat.reshape(b, k, CODE_DIM)
    indices = idx.reshape(b, k)
    return (q_st, indices, loss.reshape(()), ent.reshape(()))


# speedup vs baseline: 29.0999x; 1.0983x over previous
import functools

import jax
import jax.numpy as jnp
from jax import lax
from jax.experimental import pallas as pl
from jax.experimental.pallas import tpu as pltpu

N_CODES = 1024
CODE_DIM = 64
K_TOP = 8
INV_TEMP = 10.0


def _vq_kernel(x_ref, wt_ref, b_ref, e_ref, iota_ref,
               q_ref, idx_ref, loss_ref, ent_ref,
               usage_scr, loss_scr, *, nsteps, n_rows):
    i = pl.program_id(0)

    @pl.when(i == 0)
    def _init():
        usage_scr[...] = jnp.zeros_like(usage_scr)
        loss_scr[0, 0] = 0.0

    r = x_ref.shape[0]
    x = x_ref[...]
    z = lax.dot_general(x, wt_ref[...], (((1,), (0,)), ((), ())),
                        preferred_element_type=jnp.float32,
                        precision=lax.Precision.DEFAULT) + b_ref[...]
    e = e_ref[...]
    e2 = jnp.sum(e * e, axis=1)
    z2 = jnp.sum(z * z, axis=1, keepdims=True)
    cross = lax.dot_general(-2.0 * z, e, (((1,), (1,)), ((), ())),
                            preferred_element_type=jnp.float32,
                            precision=lax.Precision.DEFAULT)
    sq = (z2 + e2[None, :]) + cross

    BIG = jnp.float32(3e38)
    lanes = N_CODES // K_TOP
    c = [sq[:, g * lanes:(g + 1) * lanes] for g in range(K_TOP)]
    _NET = [(0, 1), (2, 3), (4, 5), (6, 7),
            (0, 2), (1, 3), (4, 6), (5, 7),
            (1, 2), (5, 6),
            (0, 4), (1, 5), (2, 6), (3, 7),
            (2, 4), (3, 5),
            (1, 2), (3, 4), (5, 6)]
    for lo_i, hi_i in _NET:
        lo = jnp.minimum(c[lo_i], c[hi_i])
        hi = jnp.maximum(c[lo_i], c[hi_i])
        c[lo_i], c[hi_i] = lo, hi
    vals = []
    for j in range(K_TOP):
        m = jnp.min(c[0], axis=1, keepdims=True)
        vals.append(m)
        if j == 0:
            mask0 = jnp.where(sq == m, 1.0, 0.0)
            idxf = lax.dot_general(mask0, iota_ref[...],
                                   (((1,), (0,)), ((), ())),
                                   preferred_element_type=jnp.float32)
            idx_ref[...] = idxf.astype(jnp.int32)
        if j < K_TOP - 1:
            eq = c[0] == m
            depth = K_TOP - j
            for lvl in range(depth - 1):
                c[lvl] = jnp.where(eq, c[lvl + 1], c[lvl])
            c[depth - 1] = jnp.where(eq, BIG, c[depth - 1])

    m_stack = jnp.concatenate(vals, axis=1)
    d_vals = jnp.sqrt(jnp.clip(m_stack, 1e-12, None))
    d0 = d_vals[:, 0:1]
    wexp = jnp.exp((d0 - d_vals) * INV_TEMP)
    denom = jnp.sum(wexp, axis=1, keepdims=True)
    thresh = m_stack[:, K_TOP - 1:K_TOP]

    dist = jnp.sqrt(jnp.clip(sq, 1e-12, None))
    sw = jnp.where(sq <= thresh,
                   jnp.exp((d0 - dist) * INV_TEMP), 0.0) / denom

    q = lax.dot_general(sw, e, (((1,), (0,)), ((), ())),
                        preferred_element_type=jnp.float32,
                        precision=lax.Precision.DEFAULT)
    q_ref[...] = q

    usage_scr[...] += jnp.sum(sw, axis=0)
    loss_scr[0, 0] += jnp.sum((z - q) ** 2)

    @pl.when(i == nsteps - 1)
    def _fini():
        loss = loss_scr[0, 0] / jnp.float32(n_rows * CODE_DIM)
        loss_ref[...] = jnp.full((1, 1), loss, jnp.float32)
        avg = usage_scr[...] * jnp.float32(1.0 / n_rows)
        ent = -jnp.sum(avg * jnp.log(avg + 1e-8))
        ent_ref[...] = jnp.full((1, 1), ent, jnp.float32)


def kernel(slot_features, W, b_lin, embed):
    b, k, d_model = slot_features.shape
    n = b * k
    x = slot_features.reshape(n, d_model)
    wt = W.T
    bb = b_lin.reshape(1, CODE_DIM)

    r = 512
    while n % r:
        r //= 2
    nsteps = n // r

    q_flat, idx, loss, ent = pl.pallas_call(
        functools.partial(_vq_kernel, nsteps=nsteps, n_rows=n),
        grid=(nsteps,),
        in_specs=[
            pl.BlockSpec((r, d_model), lambda i: (i, 0)),
            pl.BlockSpec((d_model, CODE_DIM), lambda i: (0, 0)),
            pl.BlockSpec((1, CODE_DIM), lambda i: (0, 0)),
            pl.BlockSpec((N_CODES, CODE_DIM), lambda i: (0, 0)),
            pl.BlockSpec((N_CODES, 1), lambda i: (0, 0)),
        ],
        out_specs=[
            pl.BlockSpec((r, CODE_DIM), lambda i: (i, 0)),
            pl.BlockSpec((r, 1), lambda i: (i, 0)),
            pl.BlockSpec((1, 1), lambda i: (0, 0)),
            pl.BlockSpec((1, 1), lambda i: (0, 0)),
        ],
        out_shape=[
            jax.ShapeDtypeStruct((n, CODE_DIM), jnp.float32),
            jax.ShapeDtypeStruct((n, 1), jnp.int32),
            jax.ShapeDtypeStruct((1, 1), jnp.float32),
            jax.ShapeDtypeStruct((1, 1), jnp.float32),
        ],
        scratch_shapes=[
            pltpu.VMEM((N_CODES,), jnp.float32),
            pltpu.SMEM((1, 1), jnp.float32),
        ],
        compiler_params=pltpu.CompilerParams(
            dimension_semantics=("arbitrary",)),
    )(x, wt, bb, embed,
      jnp.arange(N_CODES, dtype=jnp.float32).reshape(N_CODES, 1))

    q_st = q_flat.reshape(b, k, CODE_DIM)
    indices = idx.reshape(b, k)
    return (q_st, indices, loss.reshape(()), ent.reshape(()))
